# baseline (speedup 1.0000x reference)
import functools

import jax
import jax.numpy as jnp
from jax import lax
from jax.experimental import pallas as pl
from jax.experimental.pallas import tpu as pltpu
from jax.experimental.pallas import tpu_sc as plsc

CHUNK = 128


@functools.partial(jax.jit, static_argnums=(4, 5))
def _sc_gather(user_table, uidx, movie_table, midx, B, D):
    info = plsc.get_sparse_core_info()
    NW = info.num_cores * info.num_subcores
    b_per_w = B // NW
    n_ch = b_per_w // CHUNK
    mesh = plsc.VectorSubcoreMesh(core_axis_name="c", subcore_axis_name="s")

    @functools.partial(
        pl.kernel,
        mesh=mesh,
        compiler_params=pltpu.CompilerParams(use_tc_tiling_on_sc=False),
        out_type=(
            jax.ShapeDtypeStruct((B, D), jnp.float32),
            jax.ShapeDtypeStruct((B, D), jnp.float32),
        ),
        scratch_types=[
            pltpu.VMEM((n_ch, CHUNK), jnp.int32),
            pltpu.VMEM((n_ch, CHUNK), jnp.int32),
            pltpu.VMEM((b_per_w, D), jnp.float32),
            pltpu.VMEM((b_per_w, D), jnp.float32),
            pltpu.SemaphoreType.DMA,
        ],
    )
    def k(ut_hbm, uix_hbm, mt_hbm, mix_hbm, ue_hbm, me_hbm,
          uidx_v, midx_v, urows_v, mrows_v, sem):
        wid = lax.axis_index("s") * info.num_cores + lax.axis_index("c")
        base = wid * b_per_w
        pltpu.sync_copy(uix_hbm.at[wid], uidx_v)
        pltpu.sync_copy(mix_hbm.at[wid], midx_v)
        copies = []
        for j in range(n_ch):
            dst = pl.ds(j * CHUNK, CHUNK)
            copies.append(pltpu.async_copy(ut_hbm.at[uidx_v.at[j]], urows_v.at[dst], sem))
            copies.append(pltpu.async_copy(mt_hbm.at[midx_v.at[j]], mrows_v.at[dst], sem))
        for c in copies:
            c.wait()
        pltpu.sync_copy(urows_v, ue_hbm.at[pl.ds(base, b_per_w)])
        pltpu.sync_copy(mrows_v, me_hbm.at[pl.ds(base, b_per_w)])

    uix3 = uidx.reshape(NW, n_ch, CHUNK)
    mix3 = midx.reshape(NW, n_ch, CHUNK)
    return k(user_table, uix3, movie_table, mix3)


def _mlp_body(ue_ref, me_ref, w1u_ref, w1m_ref, b1_ref, w2_ref, b2_ref, out_ref):
    h = jnp.dot(ue_ref[...], w1u_ref[...], preferred_element_type=jnp.float32)
    h = h + jnp.dot(me_ref[...], w1m_ref[...], preferred_element_type=jnp.float32)
    h = jnp.maximum(h + b1_ref[...], 0.0)
    out_ref[...] = jnp.dot(h, w2_ref[...], preferred_element_type=jnp.float32) + b2_ref[...]


def _tc_mlp(ue, me, w1u, w1m, b1, w2, b2):
    B, D = ue.shape
    H = w1u.shape[1]
    BLK = 2048
    return pl.pallas_call(
        _mlp_body,
        grid=(B // BLK,),
        in_specs=[
            pl.BlockSpec((BLK, D), lambda i: (i, 0)),
            pl.BlockSpec((BLK, D), lambda i: (i, 0)),
            pl.BlockSpec((D, H), lambda i: (0, 0)),
            pl.BlockSpec((D, H), lambda i: (0, 0)),
            pl.BlockSpec((1, H), lambda i: (0, 0)),
            pl.BlockSpec((H, 1), lambda i: (0, 0)),
            pl.BlockSpec((1, 1), lambda i: (0, 0)),
        ],
        out_specs=pl.BlockSpec((BLK, 1), lambda i: (i, 0)),
        out_shape=jax.ShapeDtypeStruct((B, 1), jnp.float32),
    )(ue, me, w1u, w1m, b1, w2, b2)


def kernel(users, movies, user_table, movie_table, W1, b1, W2, b2):
    B = users.shape[0]
    D = user_table.shape[1]
    ue, me = _sc_gather(user_table, users.astype(jnp.int32),
                        movie_table, movies.astype(jnp.int32), B, D)
    w1t = W1.T
    out = _tc_mlp(ue, me, w1t[:D], w1t[D:],
                  b1.reshape(1, -1), W2.T, b2.reshape(1, 1))
    return out
